# baseline (speedup 1.0000x reference)
import functools

import jax
import jax.numpy as jnp
from jax import lax
from jax.experimental import pallas as pl
from jax.experimental.pallas import tpu as pltpu
from jax.experimental.pallas import tpu_sc as plsc

N = 10000
NP = 10240
E = 320000
D = 128
TE = 2 * E

NC, NS = 2, 16
NW = NC * NS
EP = E // NW
RP = NP // NS
CH = 80
TP = TE // NW
CC = 2000

_mesh = plsc.VectorSubcoreMesh(
    core_axis_name="c", subcore_axis_name="s", num_cores=NC, num_subcores=NS
)


DR = NP // D


def _deg_body(col_hbm, z128_hbm, iot_hbm, degp_hbm, deg_sh, dloc, cbuf, iot_v):
    c = lax.axis_index("c")
    s = lax.axis_index("s")
    wid = s * NC + c
    pl.when(s == 0)(lambda: pltpu.sync_copy(z128_hbm.at[pl.ds(0, DR)], deg_sh))
    pltpu.sync_copy(z128_hbm.at[pl.ds(0, DR)], dloc)
    pltpu.sync_copy(col_hbm.at[pl.ds(wid * EP, EP)], cbuf)
    pltpu.sync_copy(iot_hbm, iot_v)
    plsc.subcore_barrier()
    ones = jnp.full((16,), 1.0, jnp.float32)

    def step(i, carry):
        idx = cbuf[pl.ds(i * 16, 16)]
        r = lax.shift_right_logical(idx, 7)
        cc = lax.bitwise_and(idx, 127)
        plsc.addupdate_scatter(dloc, [r, cc], ones)
        return carry

    lax.fori_loop(0, EP // 16, step, 0)
    pltpu.sync_copy(dloc, deg_sh.at[iot_v], add=True)
    plsc.subcore_barrier()
    pl.when(s == 0)(
        lambda: pltpu.sync_copy(deg_sh, degp_hbm.at[pl.ds(c * DR, DR)]))


_deg_kernel = pl.kernel(
    _deg_body,
    out_type=jax.ShapeDtypeStruct((NC * DR, D), jnp.float32),
    mesh=_mesh,
    scratch_types=[
        pltpu.VMEM_SHARED((DR, D), jnp.float32),
        pltpu.VMEM((DR, D), jnp.float32),
        pltpu.VMEM((EP,), jnp.int32),
        pltpu.VMEM((DR,), jnp.int32),
    ],
    compiler_params=pltpu.CompilerParams(needs_layout_passes=False),
)


NCH = EP // CH


def _seg_body(row_hbm, col_hbm, y_hbm, z128_hbm, accp_hbm,
              acc_sh, ridx, cidx, msg0, msg1, sem0, sem1):
    c = lax.axis_index("c")
    s = lax.axis_index("s")
    wid = s * NC + c
    pltpu.sync_copy(z128_hbm.at[pl.ds(s * RP, RP)], acc_sh.at[pl.ds(s * RP, RP)])
    base = wid * EP
    pltpu.sync_copy(row_hbm.at[pl.ds(base, EP)], ridx)
    pltpu.sync_copy(col_hbm.at[pl.ds(base, EP)], cidx)
    plsc.subcore_barrier()
    pltpu.async_copy(y_hbm.at[ridx.at[pl.ds(0, CH)]], msg0, sem0)

    def pair(j, carry):
        i0 = 2 * j
        pltpu.async_copy(y_hbm.at[ridx.at[pl.ds((i0 + 1) * CH, CH)]],
                         msg1, sem1)
        pltpu.make_async_copy(y_hbm.at[ridx.at[pl.ds(i0 * CH, CH)]],
                              msg0, sem0).wait()
        pltpu.sync_copy(msg0, acc_sh.at[cidx.at[pl.ds(i0 * CH, CH)]], add=True)
        def _prefetch():
            pltpu.async_copy(
                y_hbm.at[ridx.at[pl.ds((i0 + 2) * CH, CH)]], msg0, sem0)

        pl.when(i0 + 2 < NCH)(_prefetch)
        pltpu.make_async_copy(y_hbm.at[ridx.at[pl.ds((i0 + 1) * CH, CH)]],
                              msg1, sem1).wait()
        pltpu.sync_copy(msg1, acc_sh.at[cidx.at[pl.ds((i0 + 1) * CH, CH)]],
                        add=True)
        return carry

    lax.fori_loop(0, NCH // 2, pair, 0)
    if NCH % 2 == 1:
        last = (NCH - 1) * CH
        pltpu.make_async_copy(y_hbm.at[ridx.at[pl.ds(last, CH)]],
                              msg0, sem0).wait()
        pltpu.sync_copy(msg0, acc_sh.at[cidx.at[pl.ds(last, CH)]], add=True)
    plsc.subcore_barrier()
    pltpu.sync_copy(acc_sh.at[pl.ds(s * RP, RP)],
                    accp_hbm.at[pl.ds(c * NP + s * RP, RP)])


_seg_kernel = pl.kernel(
    _seg_body,
    out_type=jax.ShapeDtypeStruct((NC * NP, D), jnp.float32),
    mesh=_mesh,
    scratch_types=[
        pltpu.VMEM_SHARED((NP, D), jnp.float32),
        pltpu.VMEM((EP,), jnp.int32),
        pltpu.VMEM((EP,), jnp.int32),
        pltpu.VMEM((CH, D), jnp.float32),
        pltpu.VMEM((CH, D), jnp.float32),
        pltpu.SemaphoreType.DMA,
        pltpu.SemaphoreType.DMA,
    ],
)


def _edge_body(p_hbm, q_hbm, t0_hbm, t1_hbm, out_hbm, pbuf, qbuf, t0b, t1b, ob):
    c = lax.axis_index("c")
    s = lax.axis_index("s")
    wid = s * NC + c
    pltpu.sync_copy(p_hbm, pbuf)
    pltpu.sync_copy(q_hbm, qbuf)
    base = wid * TP
    iota = lax.iota(jnp.int32, 16)

    def chunk(ch, carry):
        e0 = base + ch * CC
        pltpu.sync_copy(t0_hbm.at[pl.ds(e0, CC)], t0b)
        pltpu.sync_copy(t1_hbm.at[pl.ds(e0, CC)], t1b)

        def inner(i, icarry):
            for u in range(5):
                o16 = i * 80 + u * 16
                sv = t0b[pl.ds(o16, 16)]
                dv = t1b[pl.ds(o16, 16)]
                s2 = sv + sv
                d2 = dv + dv
                p0 = plsc.load_gather(pbuf, [s2])
                p1 = plsc.load_gather(pbuf, [s2 + 1])
                q0 = plsc.load_gather(qbuf, [d2])
                q1 = plsc.load_gather(qbuf, [d2 + 1])
                pos = o16 + iota
                plsc.store_scatter(ob, [pos], p0 + q0)
                plsc.store_scatter(ob, [pos + CC], p1 + q1)
            return icarry

        lax.fori_loop(0, CC // 80, inner, 0)
        pltpu.sync_copy(ob.at[pl.ds(0, CC)], out_hbm.at[pl.ds(e0, CC)])
        pltpu.sync_copy(ob.at[pl.ds(CC, CC)], out_hbm.at[pl.ds(TE + e0, CC)])
        return carry

    lax.fori_loop(0, TP // CC, chunk, 0)


_edge_kernel = pl.kernel(
    _edge_body,
    out_type=jax.ShapeDtypeStruct((2 * TE,), jnp.float32),
    mesh=_mesh,
    scratch_types=[
        pltpu.VMEM((2 * NP,), jnp.float32),
        pltpu.VMEM((2 * NP,), jnp.float32),
        pltpu.VMEM((CC,), jnp.int32),
        pltpu.VMEM((CC,), jnp.int32),
        pltpu.VMEM((2 * CC,), jnp.float32),
    ],
    compiler_params=pltpu.CompilerParams(needs_layout_passes=False),
)


RB = 2048


def _tc1_body(x_ref, w_ref, dvec_ref, y_ref):
    dis = lax.rsqrt(dvec_ref[...] + 1.0)
    xw = jnp.dot(x_ref[...], w_ref[...],
                 preferred_element_type=jnp.float32,
                 precision=lax.Precision.HIGHEST)
    y_ref[...] = xw * dis


def _tc2_body(accp_ref, y_ref, dvec_ref, b_ref, wc_ref, bc_ref, pq_ref):
    dis = lax.rsqrt(dvec_ref[...] + 1.0)
    acc = accp_ref[0] + accp_ref[1] + y_ref[...]
    h = jnp.maximum(dis * acc + b_ref[...], 0.0)
    pq_ref[...] = jnp.dot(h, wc_ref[...],
                          preferred_element_type=jnp.float32,
                          precision=lax.Precision.HIGHEST) + bc_ref[...]


def _tc1(x, W, dvec):
    return pl.pallas_call(
        _tc1_body,
        grid=(NP // RB,),
        in_specs=[
            pl.BlockSpec((RB, D), lambda i: (i, 0)),
            pl.BlockSpec((D, D), lambda i: (0, 0)),
            pl.BlockSpec((RB, 1), lambda i: (i, 0)),
        ],
        out_specs=pl.BlockSpec((RB, D), lambda i: (i, 0)),
        out_shape=jax.ShapeDtypeStruct((NP, D), jnp.float32),
    )(x, W, dvec)


def _tc2(accp3, y, dvec, b2, wc2, bc2):
    return pl.pallas_call(
        _tc2_body,
        grid=(NP // RB,),
        in_specs=[
            pl.BlockSpec((NC, RB, D), lambda i: (0, i, 0)),
            pl.BlockSpec((RB, D), lambda i: (i, 0)),
            pl.BlockSpec((RB, 1), lambda i: (i, 0)),
            pl.BlockSpec((1, D), lambda i: (0, 0)),
            pl.BlockSpec((D, 4), lambda i: (0, 0)),
            pl.BlockSpec((1, 4), lambda i: (0, 0)),
        ],
        out_specs=pl.BlockSpec((RB, 4), lambda i: (i, 0)),
        out_shape=jax.ShapeDtypeStruct((NP, 4), jnp.float32),
    )(accp3, y, dvec, b2, wc2, bc2)


def _neg_pairs(num_neg, n):
    k = jax.random.key(12345)
    k1, k2 = jax.random.split(k)
    src = jax.random.randint(k1, (num_neg,), 0, n, dtype=jnp.int32)
    dst = jax.random.randint(k2, (num_neg,), 0, n, dtype=jnp.int32)
    return src, dst


def kernel(x, edge_index, W, b, Wc, bc):
    row = edge_index[0]
    col = edge_index[1]
    nsrc, ndst = _neg_pairs(E, N)
    t0 = jnp.concatenate([row, nsrc])
    t1 = jnp.concatenate([col, ndst])

    z128 = jnp.zeros((NP, D), jnp.float32)
    iot = jnp.arange(DR, dtype=jnp.int32)
    xp = jnp.pad(x, ((0, NP - N), (0, 0)))

    degp = _deg_kernel(col, z128, iot)
    dvec = (degp[:DR] + degp[DR:]).reshape(NP, 1)

    y = _tc1(xp, W, dvec)

    accp = _seg_kernel(row, col, y, z128)
    accp3 = accp.reshape(NC, NP, D)

    wc2 = jnp.concatenate([Wc[:D], Wc[D:]], axis=1)
    bc2 = jnp.concatenate([bc, jnp.zeros((2,), jnp.float32)]).reshape(1, 4)
    pq = _tc2(accp3, y, dvec, b.reshape(1, D), wc2, bc2)

    p_flat = pq[:, :2].reshape(-1)
    q_flat = pq[:, 2:].reshape(-1)

    out_flat = _edge_kernel(p_flat, q_flat, t0, t1)
    return out_flat.reshape(2, TE).T


# speedup vs baseline: 1.0010x; 1.0010x over previous
import functools

import jax
import jax.numpy as jnp
from jax import lax
from jax.experimental import pallas as pl
from jax.experimental.pallas import tpu as pltpu
from jax.experimental.pallas import tpu_sc as plsc

N = 10000
NP = 10240
E = 320000
D = 128
TE = 2 * E

NC, NS = 2, 16
NW = NC * NS
EP = E // NW
RP = NP // NS
CH = 80
TP = TE // NW
CC = 2000

_mesh = plsc.VectorSubcoreMesh(
    core_axis_name="c", subcore_axis_name="s", num_cores=NC, num_subcores=NS
)


DR = NP // D


def _deg_body(col_hbm, z128_hbm, iot_hbm, degp_hbm, deg_sh, dloc, cbuf, iot_v):
    c = lax.axis_index("c")
    s = lax.axis_index("s")
    wid = s * NC + c
    pl.when(s == 0)(lambda: pltpu.sync_copy(z128_hbm.at[pl.ds(0, DR)], deg_sh))
    pltpu.sync_copy(z128_hbm.at[pl.ds(0, DR)], dloc)
    pltpu.sync_copy(col_hbm.at[pl.ds(wid * EP, EP)], cbuf)
    pltpu.sync_copy(iot_hbm, iot_v)
    plsc.subcore_barrier()
    ones = jnp.full((16,), 1.0, jnp.float32)

    def step(i, carry):
        idx = cbuf[pl.ds(i * 16, 16)]
        r = lax.shift_right_logical(idx, 7)
        cc = lax.bitwise_and(idx, 127)
        plsc.addupdate_scatter(dloc, [r, cc], ones)
        return carry

    lax.fori_loop(0, EP // 16, step, 0)
    pltpu.sync_copy(dloc, deg_sh.at[iot_v], add=True)
    plsc.subcore_barrier()
    pl.when(s == 0)(
        lambda: pltpu.sync_copy(deg_sh, degp_hbm.at[pl.ds(c * DR, DR)]))


_deg_kernel = pl.kernel(
    _deg_body,
    out_type=jax.ShapeDtypeStruct((NC * DR, D), jnp.float32),
    mesh=_mesh,
    scratch_types=[
        pltpu.VMEM_SHARED((DR, D), jnp.float32),
        pltpu.VMEM((DR, D), jnp.float32),
        pltpu.VMEM((EP,), jnp.int32),
        pltpu.VMEM((DR,), jnp.int32),
    ],
    compiler_params=pltpu.CompilerParams(needs_layout_passes=False),
)


NCH = EP // CH


def _seg_body(row_hbm, col_hbm, y_hbm, z128_hbm, accp_hbm,
              acc_sh, ridx, cidx, msg0, msg1, sem0, sem1):
    c = lax.axis_index("c")
    s = lax.axis_index("s")
    wid = s * NC + c
    pltpu.sync_copy(z128_hbm.at[pl.ds(s * RP, RP)], acc_sh.at[pl.ds(s * RP, RP)])
    base = wid * EP
    pltpu.sync_copy(row_hbm.at[pl.ds(base, EP)], ridx)
    pltpu.sync_copy(col_hbm.at[pl.ds(base, EP)], cidx)
    plsc.subcore_barrier()
    pltpu.async_copy(y_hbm.at[ridx.at[pl.ds(0, CH)]], msg0, sem0)

    def pair(j, carry):
        i0 = 2 * j
        pltpu.async_copy(y_hbm.at[ridx.at[pl.ds((i0 + 1) * CH, CH)]],
                         msg1, sem1)
        pltpu.make_async_copy(y_hbm.at[ridx.at[pl.ds(i0 * CH, CH)]],
                              msg0, sem0).wait()
        pltpu.sync_copy(msg0, acc_sh.at[cidx.at[pl.ds(i0 * CH, CH)]], add=True)
        def _prefetch():
            pltpu.async_copy(
                y_hbm.at[ridx.at[pl.ds((i0 + 2) * CH, CH)]], msg0, sem0)

        pl.when(i0 + 2 < NCH)(_prefetch)
        pltpu.make_async_copy(y_hbm.at[ridx.at[pl.ds((i0 + 1) * CH, CH)]],
                              msg1, sem1).wait()
        pltpu.sync_copy(msg1, acc_sh.at[cidx.at[pl.ds((i0 + 1) * CH, CH)]],
                        add=True)
        return carry

    lax.fori_loop(0, NCH // 2, pair, 0)
    if NCH % 2 == 1:
        last = (NCH - 1) * CH
        pltpu.make_async_copy(y_hbm.at[ridx.at[pl.ds(last, CH)]],
                              msg0, sem0).wait()
        pltpu.sync_copy(msg0, acc_sh.at[cidx.at[pl.ds(last, CH)]], add=True)
    plsc.subcore_barrier()
    pltpu.sync_copy(acc_sh.at[pl.ds(s * RP, RP)],
                    accp_hbm.at[pl.ds(c * NP + s * RP, RP)])


_seg_kernel = pl.kernel(
    _seg_body,
    out_type=jax.ShapeDtypeStruct((NC * NP, D), jnp.float32),
    mesh=_mesh,
    scratch_types=[
        pltpu.VMEM_SHARED((NP, D), jnp.float32),
        pltpu.VMEM((EP,), jnp.int32),
        pltpu.VMEM((EP,), jnp.int32),
        pltpu.VMEM((CH, D), jnp.float32),
        pltpu.VMEM((CH, D), jnp.float32),
        pltpu.SemaphoreType.DMA,
        pltpu.SemaphoreType.DMA,
    ],
)


def _edge_body(p_hbm, q_hbm, t0_hbm, t1_hbm, out_hbm, pbuf, qbuf, t0b, t1b, ob):
    c = lax.axis_index("c")
    s = lax.axis_index("s")
    wid = s * NC + c
    pltpu.sync_copy(p_hbm, pbuf)
    pltpu.sync_copy(q_hbm, qbuf)
    base = wid * TP
    iota = lax.iota(jnp.int32, 16)

    def chunk(ch, carry):
        e0 = base + ch * CC
        pltpu.sync_copy(t0_hbm.at[pl.ds(e0, CC)], t0b)
        pltpu.sync_copy(t1_hbm.at[pl.ds(e0, CC)], t1b)

        def inner(i, icarry):
            for u in range(5):
                o16 = i * 80 + u * 16
                sv = t0b[pl.ds(o16, 16)]
                dv = t1b[pl.ds(o16, 16)]
                s2 = sv + sv
                d2 = dv + dv
                p0 = plsc.load_gather(pbuf, [s2])
                p1 = plsc.load_gather(pbuf, [s2 + 1])
                q0 = plsc.load_gather(qbuf, [d2])
                q1 = plsc.load_gather(qbuf, [d2 + 1])
                pos = o16 + iota
                plsc.store_scatter(ob, [pos], p0 + q0)
                plsc.store_scatter(ob, [pos + CC], p1 + q1)
            return icarry

        lax.fori_loop(0, CC // 80, inner, 0)
        pltpu.sync_copy(ob.at[pl.ds(0, CC)], out_hbm.at[pl.ds(e0, CC)])
        pltpu.sync_copy(ob.at[pl.ds(CC, CC)], out_hbm.at[pl.ds(TE + e0, CC)])
        return carry

    lax.fori_loop(0, TP // CC, chunk, 0)


_edge_kernel = pl.kernel(
    _edge_body,
    out_type=jax.ShapeDtypeStruct((2 * TE,), jnp.float32),
    mesh=_mesh,
    scratch_types=[
        pltpu.VMEM((2 * NP,), jnp.float32),
        pltpu.VMEM((2 * NP,), jnp.float32),
        pltpu.VMEM((CC,), jnp.int32),
        pltpu.VMEM((CC,), jnp.int32),
        pltpu.VMEM((2 * CC,), jnp.float32),
    ],
    compiler_params=pltpu.CompilerParams(needs_layout_passes=False),
)


RB = 2048


def _mm_body(x_ref, w_ref, xw_ref):
    xw_ref[...] = jnp.dot(x_ref[...], w_ref[...],
                          preferred_element_type=jnp.float32,
                          precision=lax.Precision.HIGHEST)


def _scale_body(xw_ref, dvec_ref, y_ref):
    dis = lax.rsqrt(dvec_ref[...] + 1.0)
    y_ref[...] = xw_ref[...] * dis


def _tc2_body(accp_ref, y_ref, dvec_ref, b_ref, wc_ref, bc_ref, pq_ref):
    dis = lax.rsqrt(dvec_ref[...] + 1.0)
    acc = accp_ref[0] + accp_ref[1] + y_ref[...]
    h = jnp.maximum(dis * acc + b_ref[...], 0.0)
    pq_ref[...] = jnp.dot(h, wc_ref[...],
                          preferred_element_type=jnp.float32,
                          precision=lax.Precision.HIGHEST) + bc_ref[...]


def _tc_mm(x, W):
    return pl.pallas_call(
        _mm_body,
        grid=(NP // RB,),
        in_specs=[
            pl.BlockSpec((RB, D), lambda i: (i, 0)),
            pl.BlockSpec((D, D), lambda i: (0, 0)),
        ],
        out_specs=pl.BlockSpec((RB, D), lambda i: (i, 0)),
        out_shape=jax.ShapeDtypeStruct((NP, D), jnp.float32),
    )(x, W)


def _tc_scale(xw, dvec):
    return pl.pallas_call(
        _scale_body,
        grid=(NP // RB,),
        in_specs=[
            pl.BlockSpec((RB, D), lambda i: (i, 0)),
            pl.BlockSpec((RB, 1), lambda i: (i, 0)),
        ],
        out_specs=pl.BlockSpec((RB, D), lambda i: (i, 0)),
        out_shape=jax.ShapeDtypeStruct((NP, D), jnp.float32),
    )(xw, dvec)


def _tc2(accp3, y, dvec, b2, wc2, bc2):
    return pl.pallas_call(
        _tc2_body,
        grid=(NP // RB,),
        in_specs=[
            pl.BlockSpec((NC, RB, D), lambda i: (0, i, 0)),
            pl.BlockSpec((RB, D), lambda i: (i, 0)),
            pl.BlockSpec((RB, 1), lambda i: (i, 0)),
            pl.BlockSpec((1, D), lambda i: (0, 0)),
            pl.BlockSpec((D, 4), lambda i: (0, 0)),
            pl.BlockSpec((1, 4), lambda i: (0, 0)),
        ],
        out_specs=pl.BlockSpec((RB, 4), lambda i: (i, 0)),
        out_shape=jax.ShapeDtypeStruct((NP, 4), jnp.float32),
    )(accp3, y, dvec, b2, wc2, bc2)


def _neg_pairs(num_neg, n):
    k = jax.random.key(12345)
    k1, k2 = jax.random.split(k)
    src = jax.random.randint(k1, (num_neg,), 0, n, dtype=jnp.int32)
    dst = jax.random.randint(k2, (num_neg,), 0, n, dtype=jnp.int32)
    return src, dst


def kernel(x, edge_index, W, b, Wc, bc):
    row = edge_index[0]
    col = edge_index[1]
    nsrc, ndst = _neg_pairs(E, N)
    t0 = jnp.concatenate([row, nsrc])
    t1 = jnp.concatenate([col, ndst])

    z128 = jnp.zeros((NP, D), jnp.float32)
    iot = jnp.arange(DR, dtype=jnp.int32)
    xp = jnp.pad(x, ((0, NP - N), (0, 0)))

    xw = _tc_mm(xp, W)
    degp = _deg_kernel(col, z128, iot)
    dvec = (degp[:DR] + degp[DR:]).reshape(NP, 1)
    y = _tc_scale(xw, dvec)

    accp = _seg_kernel(row, col, y, z128)
    accp3 = accp.reshape(NC, NP, D)

    wc2 = jnp.concatenate([Wc[:D], Wc[D:]], axis=1)
    bc2 = jnp.concatenate([bc, jnp.zeros((2,), jnp.float32)]).reshape(1, 4)
    pq = _tc2(accp3, y, dvec, b.reshape(1, D), wc2, bc2)

    p_flat = pq[:, :2].reshape(-1)
    q_flat = pq[:, 2:].reshape(-1)

    out_flat = _edge_kernel(p_flat, q_flat, t0, t1)
    return out_flat.reshape(2, TE).T


# speedup vs baseline: 1.0042x; 1.0032x over previous
import jax
import jax.numpy as jnp
from jax import lax
from jax.experimental import pallas as pl
from jax.experimental.pallas import tpu as pltpu
from jax.experimental.pallas import tpu_sc as plsc

N = 10000
NP = 10240
E = 320000
D = 128
TE = 2 * E

NC, NS = 2, 16
NW = NC * NS
EP = E // NW
RP = NP // NS
CH = 80
TP = TE // NW
CC = 2000

_mesh = plsc.VectorSubcoreMesh(
    core_axis_name="c", subcore_axis_name="s", num_cores=NC, num_subcores=NS
)


DR = NP // D


def _deg_body(col_hbm, z128_hbm, iot_hbm, degp_hbm, deg_sh, dloc, cbuf, iot_v):
    c = lax.axis_index("c")
    s = lax.axis_index("s")
    wid = s * NC + c
    pl.when(s == 0)(lambda: pltpu.sync_copy(z128_hbm.at[pl.ds(0, DR)], deg_sh))
    pltpu.sync_copy(z128_hbm.at[pl.ds(0, DR)], dloc)
    pltpu.sync_copy(col_hbm.at[pl.ds(wid * EP, EP)], cbuf)
    pltpu.sync_copy(iot_hbm, iot_v)
    plsc.subcore_barrier()
    ones = jnp.full((16,), 1.0, jnp.float32)

    def step(i, carry):
        idx = cbuf[pl.ds(i * 16, 16)]
        r = lax.shift_right_logical(idx, 7)
        cc = lax.bitwise_and(idx, 127)
        plsc.addupdate_scatter(dloc, [r, cc], ones)
        return carry

    lax.fori_loop(0, EP // 16, step, 0)
    pltpu.sync_copy(dloc, deg_sh.at[iot_v], add=True)
    plsc.subcore_barrier()
    pl.when(s == 0)(
        lambda: pltpu.sync_copy(deg_sh, degp_hbm.at[pl.ds(c * DR, DR)]))


_deg_kernel = pl.kernel(
    _deg_body,
    out_type=jax.ShapeDtypeStruct((NC * DR, D), jnp.float32),
    mesh=_mesh,
    scratch_types=[
        pltpu.VMEM_SHARED((DR, D), jnp.float32),
        pltpu.VMEM((DR, D), jnp.float32),
        pltpu.VMEM((EP,), jnp.int32),
        pltpu.VMEM((DR,), jnp.int32),
    ],
    compiler_params=pltpu.CompilerParams(needs_layout_passes=False),
)


NCH = EP // CH


def _seg_body(row_hbm, col_hbm, y_hbm, z128_hbm, accp_hbm,
              acc_sh, ridx, cidx, msg0, msg1, sem0, sem1):
    c = lax.axis_index("c")
    s = lax.axis_index("s")
    wid = s * NC + c
    pltpu.sync_copy(z128_hbm.at[pl.ds(s * RP, RP)], acc_sh.at[pl.ds(s * RP, RP)])
    base = wid * EP
    pltpu.sync_copy(row_hbm.at[pl.ds(base, EP)], ridx)
    pltpu.sync_copy(col_hbm.at[pl.ds(base, EP)], cidx)
    plsc.subcore_barrier()
    pltpu.async_copy(y_hbm.at[ridx.at[pl.ds(0, CH)]], msg0, sem0)

    def pair(j, carry):
        i0 = 2 * j
        pltpu.async_copy(y_hbm.at[ridx.at[pl.ds((i0 + 1) * CH, CH)]],
                         msg1, sem1)
        pltpu.make_async_copy(y_hbm.at[ridx.at[pl.ds(i0 * CH, CH)]],
                              msg0, sem0).wait()
        pltpu.sync_copy(msg0, acc_sh.at[cidx.at[pl.ds(i0 * CH, CH)]], add=True)
        def _prefetch():
            pltpu.async_copy(
                y_hbm.at[ridx.at[pl.ds((i0 + 2) * CH, CH)]], msg0, sem0)

        pl.when(i0 + 2 < NCH)(_prefetch)
        pltpu.make_async_copy(y_hbm.at[ridx.at[pl.ds((i0 + 1) * CH, CH)]],
                              msg1, sem1).wait()
        pltpu.sync_copy(msg1, acc_sh.at[cidx.at[pl.ds((i0 + 1) * CH, CH)]],
                        add=True)
        return carry

    lax.fori_loop(0, NCH // 2, pair, 0)
    if NCH % 2 == 1:
        last = (NCH - 1) * CH
        pltpu.make_async_copy(y_hbm.at[ridx.at[pl.ds(last, CH)]],
                              msg0, sem0).wait()
        pltpu.sync_copy(msg0, acc_sh.at[cidx.at[pl.ds(last, CH)]], add=True)
    plsc.subcore_barrier()
    pltpu.sync_copy(acc_sh.at[pl.ds(s * RP, RP)],
                    accp_hbm.at[pl.ds(c * NP + s * RP, RP)])


_seg_kernel = pl.kernel(
    _seg_body,
    out_type=jax.ShapeDtypeStruct((NC * NP, D), jnp.float32),
    mesh=_mesh,
    scratch_types=[
        pltpu.VMEM_SHARED((NP, D), jnp.float32),
        pltpu.VMEM((EP,), jnp.int32),
        pltpu.VMEM((EP,), jnp.int32),
        pltpu.VMEM((CH, D), jnp.float32),
        pltpu.VMEM((CH, D), jnp.float32),
        pltpu.SemaphoreType.DMA,
        pltpu.SemaphoreType.DMA,
    ],
)


def _edge_body(p_hbm, q_hbm, t0_hbm, t1_hbm, out_hbm, pbuf, qbuf, t0b, t1b, ob):
    c = lax.axis_index("c")
    s = lax.axis_index("s")
    wid = s * NC + c
    pltpu.sync_copy(p_hbm, pbuf)
    pltpu.sync_copy(q_hbm, qbuf)
    base = wid * TP
    iota = lax.iota(jnp.int32, 16)

    def chunk(ch, carry):
        e0 = base + ch * CC
        pltpu.sync_copy(t0_hbm.at[pl.ds(e0, CC)], t0b)
        pltpu.sync_copy(t1_hbm.at[pl.ds(e0, CC)], t1b)

        def inner(i, icarry):
            for u in range(5):
                o16 = i * 80 + u * 16
                sv = t0b[pl.ds(o16, 16)]
                dv = t1b[pl.ds(o16, 16)]
                s2 = sv + sv
                d2 = dv + dv
                p0 = plsc.load_gather(pbuf, [s2])
                p1 = plsc.load_gather(pbuf, [s2 + 1])
                q0 = plsc.load_gather(qbuf, [d2])
                q1 = plsc.load_gather(qbuf, [d2 + 1])
                pos = o16 + iota
                plsc.store_scatter(ob, [pos], p0 + q0)
                plsc.store_scatter(ob, [pos + CC], p1 + q1)
            return icarry

        lax.fori_loop(0, CC // 80, inner, 0)
        pltpu.sync_copy(ob.at[pl.ds(0, CC)], out_hbm.at[pl.ds(e0, CC)])
        pltpu.sync_copy(ob.at[pl.ds(CC, CC)], out_hbm.at[pl.ds(TE + e0, CC)])
        return carry

    lax.fori_loop(0, TP // CC, chunk, 0)


_edge_kernel = pl.kernel(
    _edge_body,
    out_type=jax.ShapeDtypeStruct((2 * TE,), jnp.float32),
    mesh=_mesh,
    scratch_types=[
        pltpu.VMEM((2 * NP,), jnp.float32),
        pltpu.VMEM((2 * NP,), jnp.float32),
        pltpu.VMEM((CC,), jnp.int32),
        pltpu.VMEM((CC,), jnp.int32),
        pltpu.VMEM((2 * CC,), jnp.float32),
    ],
    compiler_params=pltpu.CompilerParams(needs_layout_passes=False),
)


RB = 2048


def _mm_body(x_ref, w_ref, xw_ref):
    xw_ref[...] = jnp.dot(x_ref[...], w_ref[...],
                          preferred_element_type=jnp.float32,
                          precision=lax.Precision.HIGHEST)


def _scale_body(xw_ref, dvec_ref, y_ref):
    dis = lax.rsqrt(dvec_ref[...] + 1.0)
    y_ref[...] = xw_ref[...] * dis


def _tc2_body(accp_ref, y_ref, dvec_ref, b_ref, wc_ref, bc_ref, pq_ref):
    dis = lax.rsqrt(dvec_ref[...] + 1.0)
    acc = accp_ref[0] + accp_ref[1] + y_ref[...]
    h = jnp.maximum(dis * acc + b_ref[...], 0.0)
    pq_ref[...] = jnp.dot(h, wc_ref[...],
                          preferred_element_type=jnp.float32,
                          precision=lax.Precision.HIGHEST) + bc_ref[...]


def _tc_mm(x, W):
    return pl.pallas_call(
        _mm_body,
        grid=(NP // RB,),
        in_specs=[
            pl.BlockSpec((RB, D), lambda i: (i, 0)),
            pl.BlockSpec((D, D), lambda i: (0, 0)),
        ],
        out_specs=pl.BlockSpec((RB, D), lambda i: (i, 0)),
        out_shape=jax.ShapeDtypeStruct((NP, D), jnp.float32),
    )(x, W)


def _tc_scale(xw, dvec):
    return pl.pallas_call(
        _scale_body,
        grid=(NP // RB,),
        in_specs=[
            pl.BlockSpec((RB, D), lambda i: (i, 0)),
            pl.BlockSpec((RB, 1), lambda i: (i, 0)),
        ],
        out_specs=pl.BlockSpec((RB, D), lambda i: (i, 0)),
        out_shape=jax.ShapeDtypeStruct((NP, D), jnp.float32),
    )(xw, dvec)


def _tc2(accp3, y, dvec, b2, wc2, bc2):
    return pl.pallas_call(
        _tc2_body,
        grid=(NP // RB,),
        in_specs=[
            pl.BlockSpec((NC, RB, D), lambda i: (0, i, 0)),
            pl.BlockSpec((RB, D), lambda i: (i, 0)),
            pl.BlockSpec((RB, 1), lambda i: (i, 0)),
            pl.BlockSpec((1, D), lambda i: (0, 0)),
            pl.BlockSpec((D, 4), lambda i: (0, 0)),
            pl.BlockSpec((1, 4), lambda i: (0, 0)),
        ],
        out_specs=pl.BlockSpec((RB, 4), lambda i: (i, 0)),
        out_shape=jax.ShapeDtypeStruct((NP, 4), jnp.float32),
    )(accp3, y, dvec, b2, wc2, bc2)


def _neg_pairs(num_neg, n):
    k = jax.random.key(12345)
    k1, k2 = jax.random.split(k)
    src = jax.random.randint(k1, (num_neg,), 0, n, dtype=jnp.int32)
    dst = jax.random.randint(k2, (num_neg,), 0, n, dtype=jnp.int32)
    return src, dst


def kernel(x, edge_index, W, b, Wc, bc):
    row = edge_index[0]
    col = edge_index[1]
    nsrc, ndst = _neg_pairs(E, N)
    t0 = jnp.concatenate([row, nsrc])
    t1 = jnp.concatenate([col, ndst])

    z128 = jnp.zeros((NP, D), jnp.float32)
    iot = jnp.arange(DR, dtype=jnp.int32)
    xp = jnp.pad(x, ((0, NP - N), (0, 0)))

    xw = _tc_mm(xp, W)
    degp = _deg_kernel(col, z128, iot)
    dvec = (degp[:DR] + degp[DR:]).reshape(NP, 1)
    y = _tc_scale(xw, dvec)

    accp = _seg_kernel(row, col, y, z128)
    accp3 = accp.reshape(NC, NP, D)

    wc2 = jnp.concatenate([Wc[:D], Wc[D:]], axis=1)
    bc2 = jnp.concatenate([bc, jnp.zeros((2,), jnp.float32)]).reshape(1, 4)
    pq = _tc2(accp3, y, dvec, b.reshape(1, D), wc2, bc2)

    p_flat = pq[:, :2].reshape(-1)
    q_flat = pq[:, 2:].reshape(-1)

    out_flat = _edge_kernel(p_flat, q_flat, t0, t1)
    return out_flat.reshape(2, TE).T
